# baseline (speedup 1.0000x reference)
import functools

import numpy as np

import jax
import jax.numpy as jnp
from jax import lax
from jax.experimental import pallas as pl
from jax.experimental.pallas import tpu as pltpu
from jax.experimental.pallas import tpu_sc as plsc

N_ROWS = 1000
D_OUT = 1001
D_PAD = 1024
B_TOTAL = 4096 * 20


def _fuse_body(emb_ref, w_ref, b_ref, t_ref):
    t = lax.dot_general(
        emb_ref[...], w_ref[...],
        (((1,), (1,)), ((), ())),
        preferred_element_type=jnp.float32,
    ) + b_ref[...]
    u = lax.bitcast_convert_type(t, jnp.uint32) + jnp.uint32(0x8000)
    lo = u[:, : D_PAD // 2] >> 16
    hi = u[:, D_PAD // 2 :] & jnp.uint32(0xFFFF0000)
    t_ref[...] = lax.bitcast_convert_type(lo | hi, jnp.float32)


def _fuse_table(emb, W, b):
    W_pad = jnp.pad(W, ((0, D_PAD - D_OUT), (0, 0)))
    b_pad = jnp.pad(b, (0, D_PAD - D_OUT)).reshape(1, D_PAD)
    return pl.pallas_call(
        _fuse_body,
        out_shape=jax.ShapeDtypeStruct((N_ROWS, D_PAD // 2), jnp.float32),
    )(emb, W_pad, b_pad)


def _make_gather(B, D, chunk):
    info = plsc.get_sparse_core_info()
    nc, ns = info.num_cores, info.num_subcores
    nw = nc * ns
    b_per_w = B // nw
    n_chunks = b_per_w // chunk
    assert b_per_w % chunk == 0 and chunk % 8 == 0

    mesh = plsc.VectorSubcoreMesh(core_axis_name="c", subcore_axis_name="s")

    assert n_chunks % 2 == 0

    @functools.partial(
        pl.kernel,
        out_type=jax.ShapeDtypeStruct((B, D), jnp.float32),
        mesh=mesh,
        scratch_types=[
            pltpu.VMEM((n_chunks, chunk), jnp.int32),
            pltpu.VMEM((chunk, D), jnp.float32),
            pltpu.VMEM((chunk, D), jnp.float32),
            pltpu.SemaphoreType.DMA,
            pltpu.SemaphoreType.DMA,
        ],
    )
    def gather_kernel(table_hbm, idx_hbm, out_hbm, idx_v, rows0, rows1, g0, g1):
        wid = lax.axis_index("s") * nc + lax.axis_index("c")
        base = wid * b_per_w
        pltpu.sync_copy(idx_hbm.at[pl.ds(wid * n_chunks, n_chunks)], idx_v)

        pltpu.async_copy(table_hbm.at[idx_v.at[0]], rows0, g0)
        pltpu.async_copy(table_hbm.at[idx_v.at[1]], rows1, g1)

        def pair_body(k, carry):
            i = 2 * k

            def do_half(i, rows, sem):
                pltpu.make_async_copy(
                    table_hbm.at[idx_v.at[i]], rows, sem
                ).wait()
                pltpu.sync_copy(
                    rows, out_hbm.at[pl.ds(base + i * chunk, chunk)]
                )

                @pl.when(i + 2 < n_chunks)
                def _():
                    pltpu.async_copy(
                        table_hbm.at[idx_v.at[i + 2]], rows, sem
                    )

            do_half(i, rows0, g0)
            do_half(i + 1, rows1, g1)
            return carry

        lax.fori_loop(0, n_chunks // 2, pair_body, 0)

    return gather_kernel


_CHUNK = 40
_N_ITEMS = 4096
_T_STEPS = 20
_NBLK = 128
_SLABS = 4

_B_SLAB = B_TOTAL // _SLABS
_gather = _make_gather(_B_SLAB, D_PAD // 2, _CHUNK)


def _transpose_body(in_ref, out_ref):
    half = D_PAD // 2
    y = lax.bitcast_convert_type(in_ref[...].T, jnp.uint32)
    lo = lax.bitcast_convert_type(y << 16, jnp.float32)
    hi = lax.bitcast_convert_type(y & jnp.uint32(0xFFFF0000), jnp.float32)
    out_ref[0, pl.ds(0, half)] = lo
    out_ref[0, pl.ds(half, D_OUT - half)] = hi[: D_OUT - half]


def _transpose_alias_body(in_ref, acc_ref, out_ref):
    del acc_ref
    _transpose_body(in_ref, out_ref)


def _transpose_pack(flat_slab, slab, acc):
    nb = _N_ITEMS // _SLABS // _NBLK
    grid = _T_STEPS * nb
    out_shape = jax.ShapeDtypeStruct((_T_STEPS, D_OUT, _N_ITEMS), jnp.float32)
    out_spec = pl.BlockSpec(
        (1, D_OUT, _NBLK),
        lambda g, s=slab, n=nb: (g // n, 0, s * n + g % n),
    )
    in_spec = pl.BlockSpec((_NBLK, D_PAD // 2), lambda g: (g, 0))
    params = pltpu.CompilerParams(vmem_limit_bytes=100 * 1024 * 1024)
    if acc is None:
        return pl.pallas_call(
            _transpose_body,
            grid=(grid,),
            in_specs=[in_spec],
            out_specs=out_spec,
            out_shape=out_shape,
            compiler_params=params,
        )(flat_slab)
    return pl.pallas_call(
        _transpose_alias_body,
        grid=(grid,),
        in_specs=[in_spec, pl.BlockSpec(memory_space=pl.ANY)],
        out_specs=out_spec,
        out_shape=out_shape,
        input_output_aliases={1: 0},
        compiler_params=params,
    )(flat_slab, acc)


def kernel(x, emb, W, b):
    table = _fuse_table(emb, W, b)
    n_slab = _N_ITEMS // _SLABS
    xt = x.astype(jnp.int32).T
    acc = None
    for s in range(_SLABS):
        idx_s = xt[:, s * n_slab : (s + 1) * n_slab].reshape(
            _B_SLAB // _CHUNK, _CHUNK
        )
        flat = _gather(table, idx_s)
        acc = _transpose_pack(flat, s, acc)
    return jnp.transpose(acc, (2, 0, 1))


# speedup vs baseline: 1.8280x; 1.8280x over previous
import functools

import numpy as np

import jax
import jax.numpy as jnp
from jax import lax
from jax.experimental import pallas as pl
from jax.experimental.pallas import tpu as pltpu
from jax.experimental.pallas import tpu_sc as plsc

N_ROWS = 1000
D_OUT = 1001
D_PAD = 1024
B_TOTAL = 4096 * 20


def _fuse_body(emb_ref, w_ref, b_ref, t_ref):
    t = lax.dot_general(
        emb_ref[...], w_ref[...],
        (((1,), (1,)), ((), ())),
        preferred_element_type=jnp.float32,
    ) + b_ref[...]
    u = lax.bitcast_convert_type(t, jnp.uint32) + jnp.uint32(0x8000)
    lo = u[:, : D_PAD // 2] >> 16
    hi = u[:, D_PAD // 2 :] & jnp.uint32(0xFFFF0000)
    t_ref[...] = lax.bitcast_convert_type(lo | hi, jnp.float32)


def _fuse_table(emb, W, b):
    W_pad = jnp.pad(W, ((0, D_PAD - D_OUT), (0, 0)))
    b_pad = jnp.pad(b, (0, D_PAD - D_OUT)).reshape(1, D_PAD)
    return pl.pallas_call(
        _fuse_body,
        out_shape=jax.ShapeDtypeStruct((N_ROWS, D_PAD // 2), jnp.float32),
    )(emb, W_pad, b_pad)


def _make_gather(B, D, chunk):
    info = plsc.get_sparse_core_info()
    nc, ns = info.num_cores, info.num_subcores
    nw = nc * ns
    b_per_w = B // nw
    n_chunks = b_per_w // chunk
    assert b_per_w % chunk == 0 and chunk % 8 == 0

    mesh = plsc.VectorSubcoreMesh(core_axis_name="c", subcore_axis_name="s")

    assert n_chunks % 2 == 0

    @functools.partial(
        pl.kernel,
        out_type=jax.ShapeDtypeStruct((B, D), jnp.float32),
        mesh=mesh,
        scratch_types=[
            pltpu.VMEM((n_chunks, chunk), jnp.int32),
            pltpu.VMEM((chunk, D), jnp.float32),
            pltpu.VMEM((chunk, D), jnp.float32),
            pltpu.SemaphoreType.DMA,
            pltpu.SemaphoreType.DMA,
        ],
    )
    def gather_kernel(table_hbm, idx_hbm, out_hbm, idx_v, rows0, rows1, g0, g1):
        wid = lax.axis_index("s") * nc + lax.axis_index("c")
        base = wid * b_per_w
        pltpu.sync_copy(idx_hbm.at[pl.ds(wid * n_chunks, n_chunks)], idx_v)

        pltpu.async_copy(table_hbm.at[idx_v.at[0]], rows0, g0)
        pltpu.async_copy(table_hbm.at[idx_v.at[1]], rows1, g1)

        def pair_body(k, carry):
            i = 2 * k

            def do_half(i, rows, sem):
                pltpu.make_async_copy(
                    table_hbm.at[idx_v.at[i]], rows, sem
                ).wait()
                pltpu.sync_copy(
                    rows, out_hbm.at[pl.ds(base + i * chunk, chunk)]
                )

                @pl.when(i + 2 < n_chunks)
                def _():
                    pltpu.async_copy(
                        table_hbm.at[idx_v.at[i + 2]], rows, sem
                    )

            do_half(i, rows0, g0)
            do_half(i + 1, rows1, g1)
            return carry

        lax.fori_loop(0, n_chunks // 2, pair_body, 0)

    return gather_kernel


_CHUNK = 40
_N_ITEMS = 4096
_T_STEPS = 20
_NBLK = 128
_SLABS = 4

_B_SLAB = B_TOTAL // _SLABS
_gather = _make_gather(_B_SLAB, D_PAD // 2, _CHUNK)


def _transpose_body(in_ref, out_ref):
    half = D_PAD // 2
    x = in_ref[...].reshape(_NBLK, _T_STEPS, half)
    for j in range(_T_STEPS):
        y = lax.bitcast_convert_type(x[:, j, :].T, jnp.uint32)
        lo = lax.bitcast_convert_type(y << 16, jnp.float32)
        hi = lax.bitcast_convert_type(y & jnp.uint32(0xFFFF0000), jnp.float32)
        out_ref[j, pl.ds(0, half)] = lo
        out_ref[j, pl.ds(half, D_OUT - half)] = hi[: D_OUT - half]


def _transpose_alias_body(in_ref, acc_ref, out_ref):
    del acc_ref
    _transpose_body(in_ref, out_ref)


def _transpose_pack(flat_slab, slab, acc):
    grid = _N_ITEMS // _SLABS // _NBLK
    out_shape = jax.ShapeDtypeStruct((_T_STEPS, D_OUT, _N_ITEMS), jnp.float32)
    out_spec = pl.BlockSpec(
        (_T_STEPS, D_OUT, _NBLK), lambda g, s=slab, n=grid: (0, 0, s * n + g)
    )
    in_spec = pl.BlockSpec((_NBLK * _T_STEPS, D_PAD // 2), lambda g: (g, 0))
    params = pltpu.CompilerParams(vmem_limit_bytes=100 * 1024 * 1024)
    if acc is None:
        return pl.pallas_call(
            _transpose_body,
            grid=(grid,),
            in_specs=[in_spec],
            out_specs=out_spec,
            out_shape=out_shape,
            compiler_params=params,
        )(flat_slab)
    return pl.pallas_call(
        _transpose_alias_body,
        grid=(grid,),
        in_specs=[in_spec, pl.BlockSpec(memory_space=pl.ANY)],
        out_specs=out_spec,
        out_shape=out_shape,
        input_output_aliases={1: 0},
        compiler_params=params,
    )(flat_slab, acc)


def kernel(x, emb, W, b):
    table = _fuse_table(emb, W, b)
    idx = x.reshape(_SLABS, _B_SLAB // _CHUNK, _CHUNK).astype(jnp.int32)
    acc = None
    for s in range(_SLABS):
        flat = _gather(table, idx[s])
        acc = _transpose_pack(flat, s, acc)
    return jnp.transpose(acc, (2, 0, 1))
